# baseline (speedup 1.0000x reference)
import functools

import jax
import jax.numpy as jnp
from jax import lax
from jax.experimental import pallas as pl
from jax.experimental.pallas import tpu as pltpu
from jax.experimental.pallas import tpu_sc as plsc

L = 16
IDX_ROW = 256


def _sc_embed_kernel(B, P, S, V, n_chunks, rows_per_tile,
                     table_hbm, pos_hbm, idx_hbm, out_hbm,
                     table_v, pos_v, idx_v, stage_v, sem_out, sem_idx):
  info = plsc.get_sparse_core_info()
  nc = info.num_cores
  wid = lax.axis_index("s") * nc + lax.axis_index("c")
  b0 = wid * rows_per_tile

  n_sc = S // L
  n_pairs = rows_per_tile // 2
  tail = P - (n_chunks - 1) * L
  lane = lax.iota(jnp.int32, L)
  tail_idx = lane + (n_chunks - 1) * L
  tail_mask = lane < tail

  pltpu.sync_copy(table_hbm, table_v)
  pltpu.sync_copy(pos_hbm, pos_v)
  pltpu.async_copy(idx_hbm.at[pl.ds(b0 * IDX_ROW, 2 * IDX_ROW)], idx_v.at[0],
                   sem_idx)

  def pair_body(bp, _):
    b = 2 * bp
    ibuf = lax.rem(bp, 2)
    pltpu.make_async_copy(
        idx_hbm.at[pl.ds((b0 + b) * IDX_ROW, 2 * IDX_ROW)],
        idx_v.at[ibuf], sem_idx).wait()

    @pl.when(bp < n_pairs - 1)
    def _prefetch():
      pltpu.async_copy(
          idx_hbm.at[pl.ds((b0 + b + 2) * IDX_ROW, 2 * IDX_ROW)],
          idx_v.at[1 - ibuf], sem_idx)

    idx_a = [idx_v[ibuf, pl.ds(c * L, L)] for c in range(n_chunks)]
    idx_b = [idx_v[ibuf, pl.ds(IDX_ROW + c * L, L)] for c in range(n_chunks)]

    def sc_body(sc, _):
      g = bp * n_sc + sc
      t0 = lax.rem(2 * g, 3)
      t1 = lax.rem(2 * g + 1, 3)

      for j, t in ((0, t0), (1, t1)):
        @pl.when(2 * g + j >= 3)
        def _drain(t=t):
          pltpu.make_async_copy(stage_v.at[t],
                                out_hbm.at[b0 + b + j, pl.ds(sc * L, L), :],
                                sem_out.at[t]).wait()

      @plsc.parallel_loop(0, L, unroll=2)
      def si_body(si):
        s = sc * L + si
        tab_base = s * V
        pos_base = s * P
        for c in range(n_chunks - 1):
          pv = pos_v[pl.ds(pos_base + c * L, L)]
          ga = plsc.load_gather(table_v, [idx_a[c] + tab_base])
          stage_v[t0, si, pl.ds(c * L, L)] = ga + pv
          gb = plsc.load_gather(table_v, [idx_b[c] + tab_base])
          stage_v[t1, si, pl.ds(c * L, L)] = gb + pv
        pv = pos_v[pl.ds(pos_base + (n_chunks - 1) * L, L)]
        ga = plsc.load_gather(table_v, [idx_a[n_chunks - 1] + tab_base])
        gb = plsc.load_gather(table_v, [idx_b[n_chunks - 1] + tab_base])
        si_vec = jnp.full((L,), si, jnp.int32)
        plsc.store_scatter(stage_v,
                           [jnp.full((L,), t0, jnp.int32), si_vec,
                            tail_idx], ga + pv, mask=tail_mask)
        plsc.store_scatter(stage_v,
                           [jnp.full((L,), t1, jnp.int32), si_vec,
                            tail_idx], gb + pv, mask=tail_mask)

      for j, t in ((0, t0), (1, t1)):
        pltpu.async_copy(stage_v.at[t],
                         out_hbm.at[b0 + b + j, pl.ds(sc * L, L), :],
                         sem_out.at[t])
      return 0

    lax.fori_loop(0, n_sc, sc_body, 0)
    return 0

  lax.fori_loop(0, n_pairs, pair_body, 0)

  for k in range(3):
    pltpu.make_async_copy(stage_v.at[k],
                          out_hbm.at[b0, pl.ds(0, L), :],
                          sem_out.at[k]).wait()


@functools.partial(jax.jit, static_argnums=(3, 4, 5, 6))
def _embed_pos_sc(table_t_flat, pos_t_flat, x_flat, B, P, S, V):
  n_chunks = (P + L - 1) // L
  n_tiles = 32
  rows_per_tile = B // n_tiles

  mesh = plsc.VectorSubcoreMesh(core_axis_name="c", subcore_axis_name="s")
  body = functools.partial(_sc_embed_kernel, B, P, S, V, n_chunks,
                           rows_per_tile)
  run = pl.kernel(
      body,
      out_type=jax.ShapeDtypeStruct((B, S, P), jnp.float32),
      mesh=mesh,
      compiler_params=pltpu.CompilerParams(needs_layout_passes=False),
      scratch_types=[
          pltpu.VMEM((S * V,), jnp.float32),
          pltpu.VMEM((S * P + L,), jnp.float32),
          pltpu.VMEM((2, 2 * IDX_ROW), jnp.int32),
          pltpu.VMEM((3, L, P), jnp.float32),
          pltpu.SemaphoreType.DMA((3,)),
          pltpu.SemaphoreType.DMA,
      ],
  )
  return run(table_t_flat, pos_t_flat, x_flat)


def kernel(x, x_embed_weight, pos_embed):
  B, P = x.shape
  V, S = x_embed_weight.shape
  table_t_flat = x_embed_weight.T.reshape(-1)
  pos_t_flat = pos_embed.T.reshape(-1)
  pos_t_flat = jnp.pad(pos_t_flat, (0, L))
  x_flat = jnp.pad(x, ((0, 0), (0, IDX_ROW - P))).reshape(-1)
  return _embed_pos_sc(table_t_flat, pos_t_flat, x_flat, B, P, S, V)


# speedup vs baseline: 1.3269x; 1.3269x over previous
import functools

import jax
import jax.numpy as jnp
from jax import lax
from jax.experimental import pallas as pl
from jax.experimental.pallas import tpu as pltpu
from jax.experimental.pallas import tpu_sc as plsc

L = 16
IDX_ROW = 256
POS_ROW = 224


def _sc_embed_kernel(B, P, S, V, n_chunks, rows_per_tile,
                     table_hbm, pos_hbm, idx_hbm, out_hbm,
                     table_v, pos_v, idx_v, stage_v, sem_out, sem_idx):
  info = plsc.get_sparse_core_info()
  nc = info.num_cores
  wid = lax.axis_index("s") * nc + lax.axis_index("c")
  b0 = wid * rows_per_tile

  tail = P - (n_chunks - 1) * L
  lane = lax.iota(jnp.int32, L)
  tail_idx = lane + (n_chunks - 1) * L
  tail_mask = lane < tail

  pltpu.sync_copy(table_hbm, table_v)
  pltpu.sync_copy(pos_hbm, pos_v)
  pltpu.async_copy(idx_hbm.at[pl.ds(b0 * IDX_ROW, IDX_ROW)], idx_v.at[0],
                   sem_idx)

  def b_body(b, _):
    ibuf = lax.rem(b, 2)
    pltpu.make_async_copy(idx_hbm.at[pl.ds((b0 + b) * IDX_ROW, IDX_ROW)],
                          idx_v.at[ibuf], sem_idx).wait()

    @pl.when(b < rows_per_tile - 1)
    def _prefetch():
      pltpu.async_copy(idx_hbm.at[pl.ds((b0 + b + 1) * IDX_ROW, IDX_ROW)],
                       idx_v.at[1 - ibuf], sem_idx)

    idx_chunks = [idx_v[ibuf, pl.ds(c * L, L)] for c in range(n_chunks)]

    def sc_body(sc, _):
      sbuf = lax.rem(sc, 2)
      g = b * (S // L) + sc

      @pl.when(g >= 2)
      def _drain():
        pltpu.make_async_copy(stage_v.at[sbuf],
                              out_hbm.at[b0 + b, pl.ds(sc * L, L), :],
                              sem_out.at[sbuf]).wait()

      @plsc.parallel_loop(0, L, unroll=2)
      def si_body(si):
        s = sc * L + si
        tab_base = s * V
        pos_base = s * POS_ROW
        for pr in range(n_chunks // 2):
          pvp = pos_v[pl.ds(pos_base + pr * 2 * L, 2 * L)]
          pv0, pv1 = plsc.unpack(pvp, format=plsc.PackFormat.INTERLEAVED)
          c = 2 * pr
          g0 = plsc.load_gather(table_v, [idx_chunks[c] + tab_base])
          stage_v[sbuf, si, pl.ds(c * L, L)] = g0 + pv0
          g1 = plsc.load_gather(table_v, [idx_chunks[c + 1] + tab_base])
          stage_v[sbuf, si, pl.ds((c + 1) * L, L)] = g1 + pv1
        pvp = pos_v[pl.ds(pos_base + (n_chunks - 1) * L, 2 * L)]
        pv0, _ = plsc.unpack(pvp, format=plsc.PackFormat.INTERLEAVED)
        gathered = plsc.load_gather(table_v,
                                    [idx_chunks[n_chunks - 1] + tab_base])
        plsc.store_scatter(stage_v,
                           [jnp.full((L,), sbuf, jnp.int32),
                            jnp.full((L,), si, jnp.int32), tail_idx],
                           gathered + pv0, mask=tail_mask)
      pltpu.async_copy(stage_v.at[sbuf],
                       out_hbm.at[b0 + b, pl.ds(sc * L, L), :],
                       sem_out.at[sbuf])
      return 0

    lax.fori_loop(0, S // L, sc_body, 0)
    return 0

  lax.fori_loop(0, rows_per_tile, b_body, 0)

  for sbuf in range(2):
    pltpu.make_async_copy(stage_v.at[sbuf],
                          out_hbm.at[b0, pl.ds(0, L), :],
                          sem_out.at[sbuf]).wait()


@functools.partial(jax.jit, static_argnums=(3, 4, 5, 6))
def _embed_pos_sc(table_t_flat, pos_t_flat, x_flat, B, P, S, V):
  n_chunks = (P + L - 1) // L
  n_tiles = 32
  rows_per_tile = B // n_tiles

  mesh = plsc.VectorSubcoreMesh(core_axis_name="c", subcore_axis_name="s")
  body = functools.partial(_sc_embed_kernel, B, P, S, V, n_chunks,
                           rows_per_tile)
  run = pl.kernel(
      body,
      out_type=jax.ShapeDtypeStruct((B, S, P), jnp.float32),
      mesh=mesh,
      compiler_params=pltpu.CompilerParams(needs_layout_passes=False),
      scratch_types=[
          pltpu.VMEM((S * V,), jnp.float32),
          pltpu.VMEM((pos_t_flat.shape[0],), jnp.bfloat16),
          pltpu.VMEM((2, IDX_ROW), jnp.int32),
          pltpu.VMEM((2, L, P), jnp.float32),
          pltpu.SemaphoreType.DMA((2,)),
          pltpu.SemaphoreType.DMA,
      ],
  )
  return run(table_t_flat, pos_t_flat, x_flat)


def kernel(x, x_embed_weight, pos_embed):
  B, P = x.shape
  V, S = x_embed_weight.shape
  table_t_flat = x_embed_weight.T.reshape(-1)
  n_pairs = POS_ROW // 32
  pos_t = jnp.pad(pos_embed.T, ((0, 0), (0, POS_ROW - P)))
  pos_t = pos_t.astype(jnp.bfloat16).reshape(S, n_pairs, 2, L)
  pos_t = jnp.swapaxes(pos_t, 2, 3)
  pos_t_flat = pos_t.reshape(-1)
  x_flat = jnp.pad(x, ((0, 0), (0, IDX_ROW - P))).reshape(-1)
  return _embed_pos_sc(table_t_flat, pos_t_flat, x_flat, B, P, S, V)
